# baseline (speedup 1.0000x reference)
import functools

import jax
import jax.numpy as jnp
from jax import lax
from jax.experimental import pallas as pl
from jax.experimental.pallas import tpu as pltpu
from jax.experimental.pallas import tpu_sc as plsc

N = 10000
E = 320000
D = 128
H = 128
G = 64

NC = 2
NS = 16
EPC = E // NC
EPW = E // (NC * NS)
C = 40
NCHUNK = EPW // C
NB = 5
GD = 3

ROWS_A = 640
ROWS_LAST = N - 15 * ROWS_A


def _make_sc_body(want_deg):
    def body(h_hbm, src_hbm, dst_hbm, z2_hbm, *refs):
        if want_deg:
            (agg_out, deg_out, src_v, dst_v, rows_v, ones_v, dstage,
             agg_sh, deg_sh, *sems) = refs
        else:
            (agg_out, src_v, dst_v, rows_v, agg_sh, *sems) = refs
        s_si = sems[0:NB]
        s_di = sems[NB:2 * NB]
        s_g = sems[2 * NB:3 * NB]
        s_s = sems[3 * NB:4 * NB]

        cid = lax.axis_index("c")
        sid = lax.axis_index("s")

        if want_deg:
            for j in range(ROWS_A // 16):
                dstage[pl.ds(j * 16, 16)] = jnp.zeros((16,), jnp.float32)
            for j in range(-(-C // 16)):
                ones_v[0, pl.ds(j * 16, 16)] = jnp.ones((16,), jnp.float32)

        @pl.when(sid < NS - 1)
        def _():
            r0 = sid * ROWS_A
            pltpu.sync_copy(z2_hbm.at[pl.ds(r0, ROWS_A)],
                            agg_sh.at[pl.ds(r0, ROWS_A)])
            if want_deg:
                pltpu.sync_copy(dstage, deg_sh.at[pl.ds(r0, ROWS_A)])

        @pl.when(sid == NS - 1)
        def _():
            r0 = 15 * ROWS_A
            pltpu.sync_copy(z2_hbm.at[pl.ds(r0, ROWS_LAST)],
                            agg_sh.at[pl.ds(r0, ROWS_LAST)])
            if want_deg:
                pltpu.sync_copy(dstage.at[pl.ds(0, ROWS_LAST)],
                                deg_sh.at[pl.ds(r0, ROWS_LAST)])

        plsc.subcore_barrier()

        e0 = cid * EPC + sid * EPW

        def idx_start(b, i):
            base = e0 + i * C
            pltpu.async_copy(src_hbm.at[pl.ds(base, C)], src_v.at[b], s_si[b])
            pltpu.async_copy(dst_hbm.at[pl.ds(base, C)], dst_v.at[b], s_di[b])

        def idx_wait(b):
            pltpu.make_async_copy(src_hbm.at[pl.ds(0, C)], src_v.at[b],
                                  s_si[b]).wait()
            pltpu.make_async_copy(dst_hbm.at[pl.ds(0, C)], dst_v.at[b],
                                  s_di[b]).wait()

        def gather_start(b):
            pltpu.async_copy(h_hbm.at[src_v.at[b]], rows_v.at[b], s_g[b])

        def gather_wait(b):
            pltpu.make_async_copy(h_hbm.at[src_v.at[b]], rows_v.at[b],
                                  s_g[b]).wait()

        def scatter_start(b):
            pltpu.async_copy(rows_v.at[b], agg_sh.at[dst_v.at[b]], s_s[b],
                             add=True)
            if want_deg:
                pltpu.async_copy(ones_v.at[0, pl.ds(0, C)],
                                 deg_sh.at[dst_v.at[b]], s_s[b], add=True)

        def scatter_wait(b):
            pltpu.make_async_copy(rows_v.at[b], agg_sh.at[dst_v.at[b]],
                                  s_s[b]).wait()
            if want_deg:
                pltpu.make_async_copy(ones_v.at[0, pl.ds(0, C)],
                                      deg_sh.at[dst_v.at[b]], s_s[b]).wait()

        for k in range(GD + 1):
            idx_start(k, k)
        for k in range(GD):
            idx_wait(k)
            gather_start(k)

        def outer(g, carry):
            for b in range(NB):
                i = NB * g + b
                bg = (b + GD) % NB
                bi = (b + GD + 1) % NB
                bw = (b + NB - 1) % NB

                @pl.when(i < NCHUNK)
                def _():
                    gather_wait(b)

                @pl.when(i + GD < NCHUNK)
                def _():
                    idx_wait(bg)
                    gather_start(bg)

                @pl.when(i < NCHUNK)
                def _():
                    scatter_start(b)

                @pl.when(jnp.logical_and(i >= 1, i <= NCHUNK))
                def _():
                    scatter_wait(bw)

                @pl.when(i + GD + 1 < NCHUNK)
                def _():
                    idx_start(bi, i + GD + 1)
            return carry

        n_iter = -(-(NCHUNK + 2) // NB)
        lax.fori_loop(0, n_iter, outer, 0)

        plsc.subcore_barrier()

        @pl.when(sid < NS - 1)
        def _():
            r0 = sid * ROWS_A
            pltpu.sync_copy(agg_sh.at[pl.ds(r0, ROWS_A)],
                            agg_out.at[cid, pl.ds(r0, ROWS_A)])
            if want_deg:
                pltpu.sync_copy(deg_sh.at[pl.ds(r0, ROWS_A)], dstage)
                pltpu.sync_copy(dstage,
                                deg_out.at[pl.ds(cid * N + r0, ROWS_A)])

        @pl.when(sid == NS - 1)
        def _():
            r0 = 15 * ROWS_A
            pltpu.sync_copy(agg_sh.at[pl.ds(r0, ROWS_LAST)],
                            agg_out.at[cid, pl.ds(r0, ROWS_LAST)])
            if want_deg:
                pltpu.sync_copy(deg_sh.at[pl.ds(r0, ROWS_LAST)],
                                dstage.at[pl.ds(0, ROWS_LAST)])
                pltpu.sync_copy(dstage.at[pl.ds(0, ROWS_LAST)],
                                deg_out.at[pl.ds(cid * N + r0, ROWS_LAST)])

    return body


@functools.cache
def _get_sc_agg(want_deg):
    mesh = plsc.VectorSubcoreMesh(core_axis_name="c", subcore_axis_name="s")
    out_type = [jax.ShapeDtypeStruct((NC, N, D), jnp.float32)]
    scratch = [
        pltpu.VMEM((NB, C), jnp.int32),
        pltpu.VMEM((NB, C), jnp.int32),
        pltpu.VMEM((NB, C, D), jnp.float32),
    ]
    if want_deg:
        out_type.append(jax.ShapeDtypeStruct((NC * N,), jnp.float32))
        scratch += [
            pltpu.VMEM((1, 16 * -(-C // 16)), jnp.float32),
            pltpu.VMEM((ROWS_A,), jnp.float32),
        ]
    scratch.append(pltpu.VMEM_SHARED((N, D), jnp.float32))
    if want_deg:
        scratch.append(pltpu.VMEM_SHARED((N,), jnp.float32))
    scratch += [pltpu.SemaphoreType.DMA] * (4 * NB)
    return pl.kernel(
        _make_sc_body(want_deg),
        out_type=out_type,
        mesh=mesh,
        scratch_types=scratch,
    )


BM = 1000
GRID = N // BM


def _tc_layer_body(aggp_ref, degp_ref, h_ref, wl_ref, wr_ref, b_ref, o_ref):
    ap = aggp_ref[...]
    a = ap[0] + ap[1]
    dp = degp_ref[...]
    d = dp[0] + dp[1]
    mean = a / jnp.maximum(d, 1.0)
    out = (jnp.dot(mean, wl_ref[...], preferred_element_type=jnp.float32)
           + jnp.dot(h_ref[...], wr_ref[...], preferred_element_type=jnp.float32)
           + b_ref[...])
    o_ref[...] = jnp.maximum(out, 0.0)


_tc_layer = pl.pallas_call(
    _tc_layer_body,
    grid=(GRID,),
    in_specs=[
        pl.BlockSpec((NC, BM, D), lambda i: (0, i, 0)),
        pl.BlockSpec((NC, BM, 1), lambda i: (0, i, 0)),
        pl.BlockSpec((BM, D), lambda i: (i, 0)),
        pl.BlockSpec((D, H), lambda i: (0, 0)),
        pl.BlockSpec((D, H), lambda i: (0, 0)),
        pl.BlockSpec((1, H), lambda i: (0, 0)),
    ],
    out_specs=pl.BlockSpec((BM, H), lambda i: (i, 0)),
    out_shape=jax.ShapeDtypeStruct((N, H), jnp.float32),
)


def _tc_pool_body(h_ref, bt_ref, wc1_ref, bc1_ref, wc2_ref, bc2_ref, o_ref,
                  sums, cnts):
    i = pl.program_id(0)

    @pl.when(i == 0)
    def _():
        sums[...] = jnp.zeros_like(sums)
        cnts[...] = jnp.zeros_like(cnts)

    bt = bt_ref[...]
    mask = (bt == lax.broadcasted_iota(jnp.int32, (BM, G), 1)).astype(jnp.float32)
    h = h_ref[...]
    dn = (((0,), (0,)), ((), ()))
    sums[...] += lax.dot_general(mask, h, dn, preferred_element_type=jnp.float32)
    cnts[...] += lax.dot_general(mask, jnp.ones((BM, H), jnp.float32), dn,
                                 preferred_element_type=jnp.float32)

    @pl.when(i == pl.num_programs(0) - 1)
    def _():
        g = sums[...] / jnp.maximum(cnts[...], 1.0)
        hid = jnp.maximum(
            jnp.dot(g, wc1_ref[...], preferred_element_type=jnp.float32)
            + bc1_ref[...], 0.0)
        o_ref[...] = (jnp.dot(hid, wc2_ref[...], preferred_element_type=jnp.float32)
                      + bc2_ref[...])


_tc_pool = pl.pallas_call(
    _tc_pool_body,
    grid=(GRID,),
    in_specs=[
        pl.BlockSpec((BM, H), lambda i: (i, 0)),
        pl.BlockSpec((BM, 1), lambda i: (i, 0)),
        pl.BlockSpec((H, H // 2), lambda i: (0, 0)),
        pl.BlockSpec((1, H // 2), lambda i: (0, 0)),
        pl.BlockSpec((H // 2, H), lambda i: (0, 0)),
        pl.BlockSpec((1, H), lambda i: (0, 0)),
    ],
    out_specs=pl.BlockSpec((G, H), lambda i: (0, 0)),
    out_shape=jax.ShapeDtypeStruct((G, H), jnp.float32),
    scratch_shapes=[
        pltpu.VMEM((G, H), jnp.float32),
        pltpu.VMEM((G, H), jnp.float32),
    ],
)


def kernel(x, edge_index, batch, W1l, W1r, b1, W2l, W2r, b2, W3l, W3r, b3,
           Wc1, bc1, Wc2, bc2):
    src = edge_index[0]
    dst = edge_index[1]
    z2 = jnp.zeros((N, D), jnp.float32)

    sc_agg_deg = _get_sc_agg(True)
    sc_agg = _get_sc_agg(False)

    aggp, degp = sc_agg_deg(x, src, dst, z2)
    degp_r = degp.reshape(NC, N, 1)
    h = _tc_layer(aggp, degp_r, x, W1l, W1r, b1.reshape(1, H))
    for (Wl, Wr, b) in ((W2l, W2r, b2), (W3l, W3r, b3)):
        res = sc_agg(h, src, dst, z2)
        aggp = res[0] if isinstance(res, (list, tuple)) else res
        h = _tc_layer(aggp, degp_r, h, Wl, Wr, b.reshape(1, H))

    Wc2p = jnp.zeros((H // 2, H), jnp.float32).at[:, :2].set(Wc2)
    bc2p = jnp.zeros((1, H), jnp.float32).at[0, :2].set(bc2)
    out = _tc_pool(h, batch.reshape(N, 1), Wc1, bc1.reshape(1, H // 2),
                   Wc2p, bc2p)
    return out[:, :2]


# speedup vs baseline: 1.3276x; 1.3276x over previous
import functools

import jax
import jax.numpy as jnp
from jax import lax
from jax.experimental import pallas as pl
from jax.experimental.pallas import tpu as pltpu
from jax.experimental.pallas import tpu_sc as plsc

N = 10000
E = 320000
D = 128
H = 128
G = 64

NC = 2
NS = 16
EPC = E // NC
EPW = E // (NC * NS)
C = 80
NCHUNK = EPW // C
NB = 4
GD = 3

ROWS_A = 640
ROWS_LAST = N - 15 * ROWS_A


def _make_sc_body(want_deg):
    def body(h_hbm, src_hbm, dst_hbm, z2_hbm, *refs):
        if want_deg:
            (agg_out, deg_out, src_v, dst_v, rows_v, ones_v, dstage,
             agg_sh, deg_sh, *sems) = refs
        else:
            (agg_out, src_v, dst_v, rows_v, agg_sh, *sems) = refs
        s_si = sems[0:NB]
        s_di = sems[NB:2 * NB]
        s_g = sems[2 * NB:3 * NB]
        s_s = sems[3 * NB:4 * NB]

        cid = lax.axis_index("c")
        sid = lax.axis_index("s")

        if want_deg:
            for j in range(ROWS_A // 16):
                dstage[pl.ds(j * 16, 16)] = jnp.zeros((16,), jnp.float32)
            for j in range(-(-C // 16)):
                ones_v[0, pl.ds(j * 16, 16)] = jnp.ones((16,), jnp.float32)

        @pl.when(sid < NS - 1)
        def _():
            r0 = sid * ROWS_A
            pltpu.sync_copy(z2_hbm.at[pl.ds(r0, ROWS_A)],
                            agg_sh.at[pl.ds(r0, ROWS_A)])
            if want_deg:
                pltpu.sync_copy(dstage, deg_sh.at[pl.ds(r0, ROWS_A)])

        @pl.when(sid == NS - 1)
        def _():
            r0 = 15 * ROWS_A
            pltpu.sync_copy(z2_hbm.at[pl.ds(r0, ROWS_LAST)],
                            agg_sh.at[pl.ds(r0, ROWS_LAST)])
            if want_deg:
                pltpu.sync_copy(dstage.at[pl.ds(0, ROWS_LAST)],
                                deg_sh.at[pl.ds(r0, ROWS_LAST)])

        plsc.subcore_barrier()

        e0 = cid * EPC + sid * EPW

        def src_start(b, i):
            pltpu.async_copy(src_hbm.at[pl.ds(e0 + i * C, C)], src_v.at[b],
                             s_si[b])

        def src_wait(b):
            pltpu.make_async_copy(src_hbm.at[pl.ds(0, C)], src_v.at[b],
                                  s_si[b]).wait()

        def dst_start(b, i):
            pltpu.async_copy(dst_hbm.at[pl.ds(e0 + i * C, C)], dst_v.at[b],
                             s_di[b])

        def dst_wait(b):
            pltpu.make_async_copy(dst_hbm.at[pl.ds(0, C)], dst_v.at[b],
                                  s_di[b]).wait()

        def gather_start(b):
            pltpu.async_copy(h_hbm.at[src_v.at[b]], rows_v.at[b], s_g[b])

        def gather_wait(b):
            pltpu.make_async_copy(h_hbm.at[src_v.at[b]], rows_v.at[b],
                                  s_g[b]).wait()

        def scatter_start(b):
            pltpu.async_copy(rows_v.at[b], agg_sh.at[dst_v.at[b]], s_s[b],
                             add=True)
            if want_deg:
                pltpu.async_copy(ones_v.at[0, pl.ds(0, C)],
                                 deg_sh.at[dst_v.at[b]], s_s[b], add=True)

        def scatter_wait(b):
            pltpu.make_async_copy(rows_v.at[b], agg_sh.at[dst_v.at[b]],
                                  s_s[b]).wait()
            if want_deg:
                pltpu.make_async_copy(ones_v.at[0, pl.ds(0, C)],
                                      deg_sh.at[dst_v.at[b]], s_s[b]).wait()

        for k in range(NB):
            src_start(k, k)
        for k in range(GD):
            dst_start(k, k)
        for k in range(GD):
            src_wait(k)
            gather_start(k)

        def outer(g, carry):
            for b in range(NB):
                i = NB * g + b
                bg = (b + GD) % NB

                @pl.when(i < NCHUNK)
                def _():
                    gather_wait(b)

                @pl.when(jnp.logical_and(i >= 1, i <= NCHUNK))
                def _():
                    scatter_wait(bg)

                @pl.when(i + GD < NCHUNK)
                def _():
                    src_wait(bg)
                    gather_start(bg)
                    dst_start(bg, i + GD)

                @pl.when(i + NB < NCHUNK)
                def _():
                    src_start(b, i + NB)

                @pl.when(i < NCHUNK)
                def _():
                    dst_wait(b)
                    scatter_start(b)
            return carry

        n_iter = -(-(NCHUNK + 2) // NB)
        lax.fori_loop(0, n_iter, outer, 0)

        plsc.subcore_barrier()

        @pl.when(sid < NS - 1)
        def _():
            r0 = sid * ROWS_A
            pltpu.sync_copy(agg_sh.at[pl.ds(r0, ROWS_A)],
                            agg_out.at[cid, pl.ds(r0, ROWS_A)])
            if want_deg:
                pltpu.sync_copy(deg_sh.at[pl.ds(r0, ROWS_A)], dstage)
                pltpu.sync_copy(dstage,
                                deg_out.at[pl.ds(cid * N + r0, ROWS_A)])

        @pl.when(sid == NS - 1)
        def _():
            r0 = 15 * ROWS_A
            pltpu.sync_copy(agg_sh.at[pl.ds(r0, ROWS_LAST)],
                            agg_out.at[cid, pl.ds(r0, ROWS_LAST)])
            if want_deg:
                pltpu.sync_copy(deg_sh.at[pl.ds(r0, ROWS_LAST)],
                                dstage.at[pl.ds(0, ROWS_LAST)])
                pltpu.sync_copy(dstage.at[pl.ds(0, ROWS_LAST)],
                                deg_out.at[pl.ds(cid * N + r0, ROWS_LAST)])

    return body


@functools.cache
def _get_sc_agg(want_deg):
    mesh = plsc.VectorSubcoreMesh(core_axis_name="c", subcore_axis_name="s")
    out_type = [jax.ShapeDtypeStruct((NC, N, D), jnp.float32)]
    scratch = [
        pltpu.VMEM((NB, C), jnp.int32),
        pltpu.VMEM((NB, C), jnp.int32),
        pltpu.VMEM((NB, C, D), jnp.float32),
    ]
    if want_deg:
        out_type.append(jax.ShapeDtypeStruct((NC * N,), jnp.float32))
        scratch += [
            pltpu.VMEM((1, 16 * -(-C // 16)), jnp.float32),
            pltpu.VMEM((ROWS_A,), jnp.float32),
        ]
    scratch.append(pltpu.VMEM_SHARED((N, D), jnp.float32))
    if want_deg:
        scratch.append(pltpu.VMEM_SHARED((N,), jnp.float32))
    scratch += [pltpu.SemaphoreType.DMA] * (4 * NB)
    return pl.kernel(
        _make_sc_body(want_deg),
        out_type=out_type,
        mesh=mesh,
        scratch_types=scratch,
    )


BM = 1000
GRID = N // BM


def _tc_layer_body(aggp_ref, degp_ref, h_ref, wl_ref, wr_ref, b_ref, o_ref):
    ap = aggp_ref[...]
    a = ap[0] + ap[1]
    dp = degp_ref[...]
    d = dp[0] + dp[1]
    mean = a / jnp.maximum(d, 1.0)
    out = (jnp.dot(mean, wl_ref[...], preferred_element_type=jnp.float32)
           + jnp.dot(h_ref[...], wr_ref[...], preferred_element_type=jnp.float32)
           + b_ref[...])
    o_ref[...] = jnp.maximum(out, 0.0)


_tc_layer = pl.pallas_call(
    _tc_layer_body,
    grid=(GRID,),
    in_specs=[
        pl.BlockSpec((NC, BM, D), lambda i: (0, i, 0)),
        pl.BlockSpec((NC, BM, 1), lambda i: (0, i, 0)),
        pl.BlockSpec((BM, D), lambda i: (i, 0)),
        pl.BlockSpec((D, H), lambda i: (0, 0)),
        pl.BlockSpec((D, H), lambda i: (0, 0)),
        pl.BlockSpec((1, H), lambda i: (0, 0)),
    ],
    out_specs=pl.BlockSpec((BM, H), lambda i: (i, 0)),
    out_shape=jax.ShapeDtypeStruct((N, H), jnp.float32),
)


def _tc_pool_body(h_ref, bt_ref, wc1_ref, bc1_ref, wc2_ref, bc2_ref, o_ref,
                  sums, cnts):
    i = pl.program_id(0)

    @pl.when(i == 0)
    def _():
        sums[...] = jnp.zeros_like(sums)
        cnts[...] = jnp.zeros_like(cnts)

    bt = bt_ref[...]
    mask = (bt == lax.broadcasted_iota(jnp.int32, (BM, G), 1)).astype(jnp.float32)
    h = h_ref[...]
    dn = (((0,), (0,)), ((), ()))
    sums[...] += lax.dot_general(mask, h, dn, preferred_element_type=jnp.float32)
    cnts[...] += lax.dot_general(mask, jnp.ones((BM, H), jnp.float32), dn,
                                 preferred_element_type=jnp.float32)

    @pl.when(i == pl.num_programs(0) - 1)
    def _():
        g = sums[...] / jnp.maximum(cnts[...], 1.0)
        hid = jnp.maximum(
            jnp.dot(g, wc1_ref[...], preferred_element_type=jnp.float32)
            + bc1_ref[...], 0.0)
        o_ref[...] = (jnp.dot(hid, wc2_ref[...], preferred_element_type=jnp.float32)
                      + bc2_ref[...])


_tc_pool = pl.pallas_call(
    _tc_pool_body,
    grid=(GRID,),
    in_specs=[
        pl.BlockSpec((BM, H), lambda i: (i, 0)),
        pl.BlockSpec((BM, 1), lambda i: (i, 0)),
        pl.BlockSpec((H, H // 2), lambda i: (0, 0)),
        pl.BlockSpec((1, H // 2), lambda i: (0, 0)),
        pl.BlockSpec((H // 2, H), lambda i: (0, 0)),
        pl.BlockSpec((1, H), lambda i: (0, 0)),
    ],
    out_specs=pl.BlockSpec((G, H), lambda i: (0, 0)),
    out_shape=jax.ShapeDtypeStruct((G, H), jnp.float32),
    scratch_shapes=[
        pltpu.VMEM((G, H), jnp.float32),
        pltpu.VMEM((G, H), jnp.float32),
    ],
)


def kernel(x, edge_index, batch, W1l, W1r, b1, W2l, W2r, b2, W3l, W3r, b3,
           Wc1, bc1, Wc2, bc2):
    src = edge_index[0]
    dst = edge_index[1]
    z2 = jnp.zeros((N, D), jnp.float32)

    sc_agg_deg = _get_sc_agg(True)
    sc_agg = _get_sc_agg(False)

    aggp, degp = sc_agg_deg(x, src, dst, z2)
    degp_r = degp.reshape(NC, N, 1)
    h = _tc_layer(aggp, degp_r, x, W1l, W1r, b1.reshape(1, H))
    for (Wl, Wr, b) in ((W2l, W2r, b2), (W3l, W3r, b3)):
        res = sc_agg(h, src, dst, z2)
        aggp = res[0] if isinstance(res, (list, tuple)) else res
        h = _tc_layer(aggp, degp_r, h, Wl, Wr, b.reshape(1, H))

    Wc2p = jnp.zeros((H // 2, H), jnp.float32).at[:, :2].set(Wc2)
    bc2p = jnp.zeros((1, H), jnp.float32).at[0, :2].set(bc2)
    out = _tc_pool(h, batch.reshape(N, 1), Wc1, bc1.reshape(1, H // 2),
                   Wc2p, bc2p)
    return out[:, :2]


# speedup vs baseline: 1.3610x; 1.0252x over previous
import functools

import jax
import jax.numpy as jnp
from jax import lax
from jax.experimental import pallas as pl
from jax.experimental.pallas import tpu as pltpu
from jax.experimental.pallas import tpu_sc as plsc

N = 10000
E = 320000
D = 128
H = 128
G = 64

NC = 2
NS = 16
EPC = E // NC
EPW = E // (NC * NS)
C = 80
NCHUNK = EPW // C
NB = 4
GD = 3

ROWS_A = 640
ROWS_LAST = N - 15 * ROWS_A


def _make_sc_body(want_deg):
    def body(h_hbm, src_hbm, dst_hbm, z2_hbm, *refs):
        if want_deg:
            (agg_out, deg_out, src_v, dst_v, rows_v, ones_v, dstage,
             agg_sh, deg_sh, *sems) = refs
        else:
            (agg_out, src_v, dst_v, rows_v, agg_sh, *sems) = refs
        s_si = sems[0:NB]
        s_di = sems[NB:2 * NB]
        s_g = sems[2 * NB:3 * NB]
        s_s = sems[3 * NB:4 * NB]

        cid = lax.axis_index("c")
        sid = lax.axis_index("s")

        if want_deg:
            for j in range(ROWS_A // 16):
                dstage[pl.ds(j * 16, 16)] = jnp.zeros((16,), jnp.float32)
            for j in range(-(-C // 16)):
                ones_v[0, pl.ds(j * 16, 16)] = jnp.ones((16,), jnp.float32)

        @pl.when(sid < NS - 1)
        def _():
            r0 = sid * ROWS_A
            pltpu.sync_copy(z2_hbm.at[pl.ds(r0, ROWS_A)],
                            agg_sh.at[pl.ds(r0, ROWS_A)])
            if want_deg:
                pltpu.sync_copy(dstage, deg_sh.at[pl.ds(r0, ROWS_A)])

        @pl.when(sid == NS - 1)
        def _():
            r0 = 15 * ROWS_A
            pltpu.sync_copy(z2_hbm.at[pl.ds(r0, ROWS_LAST)],
                            agg_sh.at[pl.ds(r0, ROWS_LAST)])
            if want_deg:
                pltpu.sync_copy(dstage.at[pl.ds(0, ROWS_LAST)],
                                deg_sh.at[pl.ds(r0, ROWS_LAST)])

        plsc.subcore_barrier()

        e0 = cid * EPC + sid * EPW

        def src_start(b, i):
            pltpu.async_copy(src_hbm.at[pl.ds(e0 + i * C, C)], src_v.at[b],
                             s_si[b])

        def src_wait(b):
            pltpu.make_async_copy(src_hbm.at[pl.ds(0, C)], src_v.at[b],
                                  s_si[b]).wait()

        def dst_start(b, i):
            pltpu.async_copy(dst_hbm.at[pl.ds(e0 + i * C, C)], dst_v.at[b],
                             s_di[b])

        def dst_wait(b):
            pltpu.make_async_copy(dst_hbm.at[pl.ds(0, C)], dst_v.at[b],
                                  s_di[b]).wait()

        def gather_start(b):
            pltpu.async_copy(h_hbm.at[src_v.at[b]], rows_v.at[b], s_g[b])

        def gather_wait(b):
            pltpu.make_async_copy(h_hbm.at[src_v.at[b]], rows_v.at[b],
                                  s_g[b]).wait()

        def scatter_start(b):
            pltpu.async_copy(rows_v.at[b], agg_sh.at[dst_v.at[b]], s_s[b],
                             add=True)
            if want_deg:
                pltpu.async_copy(ones_v.at[0, pl.ds(0, C)],
                                 deg_sh.at[dst_v.at[b]], s_s[b], add=True)

        def scatter_wait(b):
            pltpu.make_async_copy(rows_v.at[b], agg_sh.at[dst_v.at[b]],
                                  s_s[b]).wait()
            if want_deg:
                pltpu.make_async_copy(ones_v.at[0, pl.ds(0, C)],
                                      deg_sh.at[dst_v.at[b]], s_s[b]).wait()

        for k in range(NB):
            src_start(k, k)
        for k in range(GD):
            dst_start(k, k)
        for k in range(GD):
            src_wait(k)
            gather_start(k)

        def outer(g, carry):
            for b in range(NB):
                i = NB * g + b
                bg = (b + GD) % NB

                @pl.when(i < NCHUNK)
                def _():
                    gather_wait(b)

                @pl.when(jnp.logical_and(i >= 1, i <= NCHUNK))
                def _():
                    scatter_wait(bg)

                @pl.when(i + GD < NCHUNK)
                def _():
                    src_wait(bg)
                    gather_start(bg)
                    dst_start(bg, i + GD)

                @pl.when(i + NB < NCHUNK)
                def _():
                    src_start(b, i + NB)

                @pl.when(i < NCHUNK)
                def _():
                    dst_wait(b)
                    scatter_start(b)
            return carry

        n_iter = -(-(NCHUNK + 2) // NB)
        lax.fori_loop(0, n_iter, outer, 0)

        plsc.subcore_barrier()

        @pl.when(sid < NS - 1)
        def _():
            r0 = sid * ROWS_A
            pltpu.sync_copy(agg_sh.at[pl.ds(r0, ROWS_A)],
                            agg_out.at[cid, pl.ds(r0, ROWS_A)])
            if want_deg:
                pltpu.sync_copy(deg_sh.at[pl.ds(r0, ROWS_A)], dstage)
                pltpu.sync_copy(dstage,
                                deg_out.at[pl.ds(cid * N + r0, ROWS_A)])

        @pl.when(sid == NS - 1)
        def _():
            r0 = 15 * ROWS_A
            pltpu.sync_copy(agg_sh.at[pl.ds(r0, ROWS_LAST)],
                            agg_out.at[cid, pl.ds(r0, ROWS_LAST)])
            if want_deg:
                pltpu.sync_copy(deg_sh.at[pl.ds(r0, ROWS_LAST)],
                                dstage.at[pl.ds(0, ROWS_LAST)])
                pltpu.sync_copy(dstage.at[pl.ds(0, ROWS_LAST)],
                                deg_out.at[pl.ds(cid * N + r0, ROWS_LAST)])

    return body


@functools.cache
def _get_sc_agg(want_deg):
    mesh = plsc.VectorSubcoreMesh(core_axis_name="c", subcore_axis_name="s")
    out_type = [jax.ShapeDtypeStruct((NC, N, D), jnp.float32)]
    scratch = [
        pltpu.VMEM((NB, C), jnp.int32),
        pltpu.VMEM((NB, C), jnp.int32),
        pltpu.VMEM((NB, C, D), jnp.float32),
    ]
    if want_deg:
        out_type.append(jax.ShapeDtypeStruct((NC * N,), jnp.float32))
        scratch += [
            pltpu.VMEM((1, 16 * -(-C // 16)), jnp.float32),
            pltpu.VMEM((ROWS_A,), jnp.float32),
        ]
    scratch.append(pltpu.VMEM_SHARED((N, D), jnp.float32))
    if want_deg:
        scratch.append(pltpu.VMEM_SHARED((N,), jnp.float32))
    scratch += [pltpu.SemaphoreType.DMA] * (4 * NB)
    return pl.kernel(
        _make_sc_body(want_deg),
        out_type=out_type,
        mesh=mesh,
        scratch_types=scratch,
    )


BM = 1000
GRID = N // BM


def _tc_layer_body(aggp_ref, degp_ref, h_ref, wl_ref, wr_ref, b_ref, o_ref):
    ap = aggp_ref[...]
    a = ap[0] + ap[1]
    dp = degp_ref[...]
    d = dp[0] + dp[1]
    mean = a / jnp.maximum(d, 1.0)
    out = (jnp.dot(mean, wl_ref[...], preferred_element_type=jnp.float32)
           + jnp.dot(h_ref[...], wr_ref[...], preferred_element_type=jnp.float32)
           + b_ref[...])
    o_ref[...] = jnp.maximum(out, 0.0)


_tc_layer = pl.pallas_call(
    _tc_layer_body,
    grid=(GRID,),
    in_specs=[
        pl.BlockSpec((NC, BM, D), lambda i: (0, i, 0)),
        pl.BlockSpec((NC, BM, 1), lambda i: (0, i, 0)),
        pl.BlockSpec((BM, D), lambda i: (i, 0)),
        pl.BlockSpec((D, H), lambda i: (0, 0)),
        pl.BlockSpec((D, H), lambda i: (0, 0)),
        pl.BlockSpec((1, H), lambda i: (0, 0)),
    ],
    out_specs=pl.BlockSpec((BM, H), lambda i: (i, 0)),
    out_shape=jax.ShapeDtypeStruct((N, H), jnp.float32),
)


def _tc_pool_body(aggp_ref, degp_ref, h_ref, wl_ref, wr_ref, b_ref,
                  bt_ref, wc1_ref, bc1_ref, wc2_ref, bc2_ref, o_ref,
                  sums, cnts):
    i = pl.program_id(0)

    @pl.when(i == 0)
    def _():
        sums[...] = jnp.zeros_like(sums)
        cnts[...] = jnp.zeros_like(cnts)

    ap = aggp_ref[...]
    a = ap[0] + ap[1]
    dp = degp_ref[...]
    d = dp[0] + dp[1]
    mean = a / jnp.maximum(d, 1.0)
    h = jnp.maximum(
        jnp.dot(mean, wl_ref[...], preferred_element_type=jnp.float32)
        + jnp.dot(h_ref[...], wr_ref[...], preferred_element_type=jnp.float32)
        + b_ref[...], 0.0)

    bt = bt_ref[...]
    mask = (bt == lax.broadcasted_iota(jnp.int32, (BM, G), 1)).astype(jnp.float32)
    dn = (((0,), (0,)), ((), ()))
    sums[...] += lax.dot_general(mask, h, dn, preferred_element_type=jnp.float32)
    cnts[...] += lax.dot_general(mask, jnp.ones((BM, H), jnp.float32), dn,
                                 preferred_element_type=jnp.float32)

    @pl.when(i == pl.num_programs(0) - 1)
    def _():
        g = sums[...] / jnp.maximum(cnts[...], 1.0)
        hid = jnp.maximum(
            jnp.dot(g, wc1_ref[...], preferred_element_type=jnp.float32)
            + bc1_ref[...], 0.0)
        o_ref[...] = (jnp.dot(hid, wc2_ref[...], preferred_element_type=jnp.float32)
                      + bc2_ref[...])


_tc_pool = pl.pallas_call(
    _tc_pool_body,
    grid=(GRID,),
    in_specs=[
        pl.BlockSpec((NC, BM, D), lambda i: (0, i, 0)),
        pl.BlockSpec((NC, BM, 1), lambda i: (0, i, 0)),
        pl.BlockSpec((BM, D), lambda i: (i, 0)),
        pl.BlockSpec((D, H), lambda i: (0, 0)),
        pl.BlockSpec((D, H), lambda i: (0, 0)),
        pl.BlockSpec((1, H), lambda i: (0, 0)),
        pl.BlockSpec((BM, 1), lambda i: (i, 0)),
        pl.BlockSpec((H, H // 2), lambda i: (0, 0)),
        pl.BlockSpec((1, H // 2), lambda i: (0, 0)),
        pl.BlockSpec((H // 2, H), lambda i: (0, 0)),
        pl.BlockSpec((1, H), lambda i: (0, 0)),
    ],
    out_specs=pl.BlockSpec((G, H), lambda i: (0, 0)),
    out_shape=jax.ShapeDtypeStruct((G, H), jnp.float32),
    scratch_shapes=[
        pltpu.VMEM((G, H), jnp.float32),
        pltpu.VMEM((G, H), jnp.float32),
    ],
)


def kernel(x, edge_index, batch, W1l, W1r, b1, W2l, W2r, b2, W3l, W3r, b3,
           Wc1, bc1, Wc2, bc2):
    src = edge_index[0]
    dst = edge_index[1]
    z2 = jnp.zeros((N, D), jnp.float32)

    sc_agg_deg = _get_sc_agg(True)
    sc_agg = _get_sc_agg(False)

    aggp, degp = sc_agg_deg(x, src, dst, z2)
    degp_r = degp.reshape(NC, N, 1)
    h = _tc_layer(aggp, degp_r, x, W1l, W1r, b1.reshape(1, H))

    res = sc_agg(h, src, dst, z2)
    aggp = res[0] if isinstance(res, (list, tuple)) else res
    h = _tc_layer(aggp, degp_r, h, W2l, W2r, b2.reshape(1, H))

    res = sc_agg(h, src, dst, z2)
    aggp = res[0] if isinstance(res, (list, tuple)) else res

    Wc2p = jnp.zeros((H // 2, H), jnp.float32).at[:, :2].set(Wc2)
    bc2p = jnp.zeros((1, H), jnp.float32).at[0, :2].set(bc2)
    out = _tc_pool(aggp, degp_r, h, W3l, W3r, b3.reshape(1, H),
                   batch.reshape(N, 1), Wc1, bc1.reshape(1, H // 2),
                   Wc2p, bc2p)
    return out[:, :2]
